# baseline (speedup 1.0000x reference)
import functools

import jax
import jax.numpy as jnp
import numpy as np
from jax import lax
from jax.experimental import pallas as pl
from jax.experimental.pallas import tpu as pltpu
from jax.experimental.pallas import tpu_sc as plsc

N = 10000
E = 160000
D = 128
OUT = 128
DP = 32
KRON = 400

NC = 2
NS = 16
NW = NC * NS
CH = 128
NCHUNK = E // CH
CHUNKS_PER_CORE = NCHUNK // NC
NP = 10240
ROWS_PER_TILE = NP // NS


def _node_proj_body(x_ref, w_ref, b_ref, g_ref, be_ref, o_ref):
    y = jnp.dot(x_ref[...], w_ref[...], preferred_element_type=jnp.float32)
    y = y + b_ref[...]
    mu = jnp.sum(y, axis=1, keepdims=True) * (1.0 / 20.0)
    var = jnp.sum(y * y, axis=1, keepdims=True) * (1.0 / 20.0) - mu * mu
    h = (y - mu) * lax.rsqrt(var + 1e-5) * g_ref[...] + be_ref[...]
    o_ref[...] = jnp.maximum(h, 0.0).astype(jnp.bfloat16)


def _node_proj(x, w1p, b1p, g1p, be1p):
    blk = 2000
    grid = N // blk
    return pl.pallas_call(
        _node_proj_body,
        grid=(grid,),
        in_specs=[
            pl.BlockSpec((blk, D), lambda i: (i, 0)),
            pl.BlockSpec((D, DP), lambda i: (0, 0)),
            pl.BlockSpec((1, DP), lambda i: (0, 0)),
            pl.BlockSpec((1, DP), lambda i: (0, 0)),
            pl.BlockSpec((1, DP), lambda i: (0, 0)),
        ],
        out_specs=pl.BlockSpec((blk, DP), lambda i: (i, 0)),
        out_shape=jax.ShapeDtypeStruct((N, DP), jnp.bfloat16),
    )(x, w1p, b1p, g1p, be1p)


_MESH = plsc.VectorSubcoreMesh(
    core_axis_name="c", subcore_axis_name="s", num_cores=NC, num_subcores=NS)


@functools.partial(
    pl.kernel,
    out_type=(
        jax.ShapeDtypeStruct((E, DP), jnp.bfloat16),
        jax.ShapeDtypeStruct((E, DP), jnp.bfloat16),
    ),
    mesh=_MESH,
    scratch_types=[
        pltpu.VMEM((CH,), jnp.int32),
        pltpu.VMEM((CH,), jnp.int32),
        pltpu.VMEM((CH, DP), jnp.bfloat16),
        pltpu.VMEM((CH, DP), jnp.bfloat16),
        pltpu.SemaphoreType.DMA,
        pltpu.SemaphoreType.DMA,
    ],
    compiler_params=pltpu.CompilerParams(use_tc_tiling_on_sc=False),
)
def _gather_sc(h_hbm, eis_hbm, eid_hbm, srcg_hbm, dstg_hbm,
               idxs_v, idxd_v, rows_s, rows_d, sem_s, sem_d):
    c = lax.axis_index("c")
    s = lax.axis_index("s")
    wid = s * NC + c

    def body(t, carry):
        ch = wid + t * NW

        @pl.when(ch < NCHUNK)
        def _():
            off = pl.multiple_of(ch * CH, CH)
            pltpu.sync_copy(eis_hbm.at[pl.ds(off, CH)], idxs_v)
            pltpu.sync_copy(eid_hbm.at[pl.ds(off, CH)], idxd_v)
            cps = pltpu.async_copy(h_hbm.at[idxs_v], rows_s, sem_s)
            cpd = pltpu.async_copy(h_hbm.at[idxd_v], rows_d, sem_d)
            cps.wait()
            cpd.wait()
            pltpu.sync_copy(rows_s, srcg_hbm.at[pl.ds(off, CH)])
            pltpu.sync_copy(rows_d, dstg_hbm.at[pl.ds(off, CH)])

        return carry

    lax.fori_loop(0, (NCHUNK + NW - 1) // NW, body, 0)


def _edge_body(srcg_ref, dstg_ref, r_ref, s_ref, w2_ref, b2_ref, g2_ref,
               be2_ref, o_ref):
    a = jnp.dot(srcg_ref[...], r_ref[...],
                preferred_element_type=jnp.float32)
    b = jnp.dot(dstg_ref[...], s_ref[...],
                preferred_element_type=jnp.float32)
    kron = (a * b).astype(jnp.bfloat16)
    y = jnp.dot(kron, w2_ref[...], preferred_element_type=jnp.float32)
    y = y + b2_ref[...]
    mu = jnp.mean(y, axis=1, keepdims=True)
    var = jnp.mean(y * y, axis=1, keepdims=True) - mu * mu
    h = (y - mu) * lax.rsqrt(var + 1e-5) * g2_ref[...] + be2_ref[...]
    o_ref[...] = jnp.maximum(h, 0.0)


def _edge_mlp(srcg, dstg, rmat, smat, w2, b2, g2, be2):
    blk = 1280
    grid = E // blk
    return pl.pallas_call(
        _edge_body,
        grid=(grid,),
        in_specs=[
            pl.BlockSpec((blk, DP), lambda i: (i, 0)),
            pl.BlockSpec((blk, DP), lambda i: (i, 0)),
            pl.BlockSpec((DP, KRON), lambda i: (0, 0)),
            pl.BlockSpec((DP, KRON), lambda i: (0, 0)),
            pl.BlockSpec((KRON, OUT), lambda i: (0, 0)),
            pl.BlockSpec((1, OUT), lambda i: (0, 0)),
            pl.BlockSpec((1, OUT), lambda i: (0, 0)),
            pl.BlockSpec((1, OUT), lambda i: (0, 0)),
        ],
        out_specs=pl.BlockSpec((blk, OUT), lambda i: (i, 0)),
        out_shape=jax.ShapeDtypeStruct((E, OUT), jnp.float32),
    )(srcg, dstg, rmat, smat, w2, b2, g2, be2)


@functools.partial(
    pl.kernel,
    out_type=(
        jax.ShapeDtypeStruct((NP, OUT), jnp.float32),
        jax.ShapeDtypeStruct((NP, OUT), jnp.float32),
    ),
    mesh=_MESH,
    scratch_types=[
        pltpu.VMEM_SHARED((NP, OUT), jnp.float32),
        pltpu.VMEM((CH, OUT), jnp.float32),
        pltpu.VMEM((CH, OUT), jnp.float32),
        pltpu.VMEM((CH,), jnp.int32),
    ],
)
def _scatter_sc(m_hbm, eid_hbm, zrows_hbm, p0_hbm, p1_hbm, acc, zbuf, mv, idxv):
    c = lax.axis_index("c")
    s = lax.axis_index("s")
    pltpu.sync_copy(zrows_hbm, zbuf)
    for j in range(ROWS_PER_TILE // CH):
        pltpu.sync_copy(zbuf, acc.at[pl.ds(s * ROWS_PER_TILE + j * CH, CH)])
    plsc.subcore_barrier()

    def body(t, carry):
        k = s + t * NS

        @pl.when(k < CHUNKS_PER_CORE)
        def _():
            ch = c * CHUNKS_PER_CORE + k
            off = pl.multiple_of(ch * CH, CH)
            pltpu.sync_copy(eid_hbm.at[pl.ds(off, CH)], idxv)
            pltpu.sync_copy(m_hbm.at[pl.ds(off, CH)], mv)
            pltpu.sync_copy(mv, acc.at[idxv], add=True)

        return carry

    lax.fori_loop(0, (CHUNKS_PER_CORE + NS - 1) // NS, body, 0)
    plsc.subcore_barrier()
    for j in range(ROWS_PER_TILE // CH):
        row = s * ROWS_PER_TILE + j * CH
        pltpu.sync_copy(acc.at[pl.ds(row, CH)], zbuf)

        @pl.when(c == 0)
        def _():
            pltpu.sync_copy(zbuf, p0_hbm.at[pl.ds(row, CH)])

        @pl.when(c == 1)
        def _():
            pltpu.sync_copy(zbuf, p1_hbm.at[pl.ds(row, CH)])


def _combine_body(p0_ref, p1_ref, o_ref):
    o_ref[...] = p0_ref[...] + p1_ref[...]


def _combine(p0, p1):
    blk = 2000
    grid = N // blk
    return pl.pallas_call(
        _combine_body,
        grid=(grid,),
        in_specs=[
            pl.BlockSpec((blk, OUT), lambda i: (i, 0)),
            pl.BlockSpec((blk, OUT), lambda i: (i, 0)),
        ],
        out_specs=pl.BlockSpec((blk, OUT), lambda i: (i, 0)),
        out_shape=jax.ShapeDtypeStruct((N, OUT), jnp.float32),
    )(p0, p1)


def _build_rs():
    r = np.zeros((DP, KRON), np.float32)
    s = np.zeros((DP, KRON), np.float32)
    for a in range(20):
        for k in range(20):
            r[a, a * 20 + k] = 1.0
            s[k, a * 20 + k] = 1.0
    return r, s


_R_NP, _S_NP = _build_rs()


def kernel(node_feat, edge_index, W1, b1, g1, be1, W2, b2, g2, be2):
    w1p = jnp.pad(W1, ((0, 0), (0, DP - 20)))
    b1p = jnp.pad(b1, (0, DP - 20)).reshape(1, DP)
    g1p = jnp.pad(g1, (0, DP - 20)).reshape(1, DP)
    be1p = jnp.pad(be1, (0, DP - 20)).reshape(1, DP)
    ei_src = edge_index[0]
    ei_dst = edge_index[1]
    zrows = jnp.zeros((CH, OUT), jnp.float32)

    h32 = _node_proj(node_feat, w1p, b1p, g1p, be1p)
    srcg, dstg = _gather_sc(h32, ei_src, ei_dst)
    m = _edge_mlp(srcg, dstg,
                  jnp.asarray(_R_NP, dtype=jnp.bfloat16),
                  jnp.asarray(_S_NP, dtype=jnp.bfloat16),
                  W2.astype(jnp.bfloat16),
                  b2.reshape(1, OUT), g2.reshape(1, OUT), be2.reshape(1, OUT))
    p0, p1 = _scatter_sc(m, ei_dst, zrows)
    return _combine(p0, p1)


# speedup vs baseline: 1.3328x; 1.3328x over previous
import functools

import jax
import jax.numpy as jnp
import numpy as np
from jax import lax
from jax.experimental import pallas as pl
from jax.experimental.pallas import tpu as pltpu
from jax.experimental.pallas import tpu_sc as plsc

N = 10000
E = 160000
D = 128
OUT = 128
DP = 32
KRON = 400

NC = 2
NS = 16
NW = NC * NS
CH = 128
NCHUNK = E // CH
CHUNKS_PER_CORE = NCHUNK // NC
NP = 10240
ROWS_PER_TILE = NP // NS


def _node_proj_body(x_ref, w_ref, b_ref, g_ref, be_ref, o_ref):
    y = jnp.dot(x_ref[...], w_ref[...], preferred_element_type=jnp.float32)
    y = y + b_ref[...]
    mu = jnp.sum(y, axis=1, keepdims=True) * (1.0 / 20.0)
    var = jnp.sum(y * y, axis=1, keepdims=True) * (1.0 / 20.0) - mu * mu
    h = (y - mu) * lax.rsqrt(var + 1e-5) * g_ref[...] + be_ref[...]
    o_ref[...] = jnp.maximum(h, 0.0)


def _node_proj(x, w1p, b1p, g1p, be1p):
    blk = 2000
    grid = N // blk
    return pl.pallas_call(
        _node_proj_body,
        grid=(grid,),
        in_specs=[
            pl.BlockSpec((blk, D), lambda i: (i, 0)),
            pl.BlockSpec((D, DP), lambda i: (0, 0)),
            pl.BlockSpec((1, DP), lambda i: (0, 0)),
            pl.BlockSpec((1, DP), lambda i: (0, 0)),
            pl.BlockSpec((1, DP), lambda i: (0, 0)),
        ],
        out_specs=pl.BlockSpec((blk, DP), lambda i: (i, 0)),
        out_shape=jax.ShapeDtypeStruct((N, DP), jnp.float32),
    )(x, w1p, b1p, g1p, be1p)


_MESH = plsc.VectorSubcoreMesh(
    core_axis_name="c", subcore_axis_name="s", num_cores=NC, num_subcores=NS)


@functools.partial(
    pl.kernel,
    out_type=jax.ShapeDtypeStruct((E, OUT), jnp.float32),
    mesh=_MESH,
    scratch_types=[
        pltpu.VMEM((CH,), jnp.int32),
        pltpu.VMEM((CH,), jnp.int32),
        pltpu.VMEM((CH, DP), jnp.float32),
        pltpu.VMEM((CH, DP), jnp.float32),
        pltpu.SemaphoreType.DMA,
        pltpu.SemaphoreType.DMA,
    ],
    compiler_params=pltpu.CompilerParams(use_tc_tiling_on_sc=False),
)
def _gather_sc(h_hbm, eis_hbm, eid_hbm, g_hbm,
               idxs_v, idxd_v, rows_s, rows_d, sem_s, sem_d):
    c = lax.axis_index("c")
    s = lax.axis_index("s")
    wid = s * NC + c

    def body(t, carry):
        ch = wid + t * NW

        @pl.when(ch < NCHUNK)
        def _():
            off = pl.multiple_of(ch * CH, CH)
            pltpu.sync_copy(eis_hbm.at[pl.ds(off, CH)], idxs_v)
            pltpu.sync_copy(eid_hbm.at[pl.ds(off, CH)], idxd_v)
            cps = pltpu.async_copy(h_hbm.at[idxs_v], rows_s, sem_s)
            cpd = pltpu.async_copy(h_hbm.at[idxd_v], rows_d, sem_d)
            cps.wait()
            cpd.wait()
            pltpu.sync_copy(rows_s, g_hbm.at[pl.ds(off, CH), pl.ds(0, DP)])
            pltpu.sync_copy(rows_d, g_hbm.at[pl.ds(off, CH), pl.ds(DP, DP)])

        return carry

    lax.fori_loop(0, (NCHUNK + NW - 1) // NW, body, 0)


def _edge_body(g_ref, r_ref, s_ref, w2_ref, b2_ref, g2_ref,
               be2_ref, o_ref):
    g = g_ref[...].astype(jnp.bfloat16)
    a = jnp.dot(g[:, :DP], r_ref[...], preferred_element_type=jnp.float32)
    b = jnp.dot(g[:, DP:2 * DP], s_ref[...],
                preferred_element_type=jnp.float32)
    kron = (a * b).astype(jnp.bfloat16)
    y = jnp.dot(kron, w2_ref[...], preferred_element_type=jnp.float32)
    y = y + b2_ref[...]
    mu = jnp.mean(y, axis=1, keepdims=True)
    var = jnp.mean(y * y, axis=1, keepdims=True) - mu * mu
    h = (y - mu) * lax.rsqrt(var + 1e-5) * g2_ref[...] + be2_ref[...]
    o_ref[...] = jnp.maximum(h, 0.0)


def _edge_mlp(g, rmat, smat, w2, b2, g2, be2):
    blk = 1280
    grid = E // blk
    return pl.pallas_call(
        _edge_body,
        grid=(grid,),
        in_specs=[
            pl.BlockSpec((blk, OUT), lambda i: (i, 0)),
            pl.BlockSpec((DP, KRON), lambda i: (0, 0)),
            pl.BlockSpec((DP, KRON), lambda i: (0, 0)),
            pl.BlockSpec((KRON, OUT), lambda i: (0, 0)),
            pl.BlockSpec((1, OUT), lambda i: (0, 0)),
            pl.BlockSpec((1, OUT), lambda i: (0, 0)),
            pl.BlockSpec((1, OUT), lambda i: (0, 0)),
        ],
        out_specs=pl.BlockSpec((blk, OUT), lambda i: (i, 0)),
        out_shape=jax.ShapeDtypeStruct((E, OUT), jnp.float32),
    )(g, rmat, smat, w2, b2, g2, be2)


@functools.partial(
    pl.kernel,
    out_type=(
        jax.ShapeDtypeStruct((NP, OUT), jnp.float32),
        jax.ShapeDtypeStruct((NP, OUT), jnp.float32),
    ),
    mesh=_MESH,
    scratch_types=[
        pltpu.VMEM_SHARED((NP, OUT), jnp.float32),
        pltpu.VMEM((CH, OUT), jnp.float32),
        pltpu.VMEM((CH, OUT), jnp.float32),
        pltpu.VMEM((CH,), jnp.int32),
    ],
)
def _scatter_sc(m_hbm, eid_hbm, zrows_hbm, p0_hbm, p1_hbm, acc, zbuf, mv, idxv):
    c = lax.axis_index("c")
    s = lax.axis_index("s")
    pltpu.sync_copy(zrows_hbm, zbuf)
    for j in range(ROWS_PER_TILE // CH):
        pltpu.sync_copy(zbuf, acc.at[pl.ds(s * ROWS_PER_TILE + j * CH, CH)])
    plsc.subcore_barrier()

    def body(t, carry):
        k = s + t * NS

        @pl.when(k < CHUNKS_PER_CORE)
        def _():
            ch = c * CHUNKS_PER_CORE + k
            off = pl.multiple_of(ch * CH, CH)
            pltpu.sync_copy(eid_hbm.at[pl.ds(off, CH)], idxv)
            pltpu.sync_copy(m_hbm.at[pl.ds(off, CH)], mv)
            pltpu.sync_copy(mv, acc.at[idxv], add=True)

        return carry

    lax.fori_loop(0, (CHUNKS_PER_CORE + NS - 1) // NS, body, 0)
    plsc.subcore_barrier()
    for j in range(ROWS_PER_TILE // CH):
        row = s * ROWS_PER_TILE + j * CH
        pltpu.sync_copy(acc.at[pl.ds(row, CH)], zbuf)

        @pl.when(c == 0)
        def _():
            pltpu.sync_copy(zbuf, p0_hbm.at[pl.ds(row, CH)])

        @pl.when(c == 1)
        def _():
            pltpu.sync_copy(zbuf, p1_hbm.at[pl.ds(row, CH)])


def _combine_body(p0_ref, p1_ref, o_ref):
    o_ref[...] = p0_ref[...] + p1_ref[...]


def _combine(p0, p1):
    blk = 2000
    grid = N // blk
    return pl.pallas_call(
        _combine_body,
        grid=(grid,),
        in_specs=[
            pl.BlockSpec((blk, OUT), lambda i: (i, 0)),
            pl.BlockSpec((blk, OUT), lambda i: (i, 0)),
        ],
        out_specs=pl.BlockSpec((blk, OUT), lambda i: (i, 0)),
        out_shape=jax.ShapeDtypeStruct((N, OUT), jnp.float32),
    )(p0, p1)


def _build_rs():
    r = np.zeros((DP, KRON), np.float32)
    s = np.zeros((DP, KRON), np.float32)
    for a in range(20):
        for k in range(20):
            r[a, a * 20 + k] = 1.0
            s[k, a * 20 + k] = 1.0
    return r, s


_R_NP, _S_NP = _build_rs()


def kernel(node_feat, edge_index, W1, b1, g1, be1, W2, b2, g2, be2):
    w1p = jnp.pad(W1, ((0, 0), (0, DP - 20)))
    b1p = jnp.pad(b1, (0, DP - 20)).reshape(1, DP)
    g1p = jnp.pad(g1, (0, DP - 20)).reshape(1, DP)
    be1p = jnp.pad(be1, (0, DP - 20)).reshape(1, DP)
    ei_src = edge_index[0]
    ei_dst = edge_index[1]
    zrows = jnp.zeros((CH, OUT), jnp.float32)

    h32 = _node_proj(node_feat, w1p, b1p, g1p, be1p)
    g = _gather_sc(h32, ei_src, ei_dst)
    m = _edge_mlp(g,
                  jnp.asarray(_R_NP, dtype=jnp.bfloat16),
                  jnp.asarray(_S_NP, dtype=jnp.bfloat16),
                  W2.astype(jnp.bfloat16),
                  b2.reshape(1, OUT), g2.reshape(1, OUT), be2.reshape(1, OUT))
    p0, p1 = _scatter_sc(m, ei_dst, zrows)
    return _combine(p0, p1)


# speedup vs baseline: 1.7543x; 1.3163x over previous
import functools

import jax
import jax.numpy as jnp
import numpy as np
from jax import lax
from jax.experimental import pallas as pl
from jax.experimental.pallas import tpu as pltpu
from jax.experimental.pallas import tpu_sc as plsc

N = 10000
E = 160000
D = 128
OUT = 128
DP = 32
KRON = 400

NC = 2
NS = 16
NW = NC * NS
CH = 128
NCHUNK = E // CH
CHUNKS_PER_CORE = NCHUNK // NC
NP = 10240
ROWS_PER_TILE = NP // NS


def _node_proj_body(x_ref, w_ref, b_ref, g_ref, be_ref, o_ref):
    y = jnp.dot(x_ref[...], w_ref[...], preferred_element_type=jnp.float32)
    y = y + b_ref[...]
    mu = jnp.sum(y, axis=1, keepdims=True) * (1.0 / 20.0)
    var = jnp.sum(y * y, axis=1, keepdims=True) * (1.0 / 20.0) - mu * mu
    h = (y - mu) * lax.rsqrt(var + 1e-5) * g_ref[...] + be_ref[...]
    o_ref[...] = jnp.maximum(h, 0.0)


def _node_proj(x, w1p, b1p, g1p, be1p):
    blk = 2000
    grid = N // blk
    return pl.pallas_call(
        _node_proj_body,
        grid=(grid,),
        in_specs=[
            pl.BlockSpec((blk, D), lambda i: (i, 0)),
            pl.BlockSpec((D, DP), lambda i: (0, 0)),
            pl.BlockSpec((1, DP), lambda i: (0, 0)),
            pl.BlockSpec((1, DP), lambda i: (0, 0)),
            pl.BlockSpec((1, DP), lambda i: (0, 0)),
        ],
        out_specs=pl.BlockSpec((blk, DP), lambda i: (i, 0)),
        out_shape=jax.ShapeDtypeStruct((N, DP), jnp.float32),
    )(x, w1p, b1p, g1p, be1p)


_MESH = plsc.VectorSubcoreMesh(
    core_axis_name="c", subcore_axis_name="s", num_cores=NC, num_subcores=NS)


W_CHUNKS = NCHUNK // NW
SS = 6
_SUPERSTEPS = [(k, min(SS, W_CHUNKS - k)) for k in range(0, W_CHUNKS, SS)]


@functools.partial(
    pl.kernel,
    out_type=jax.ShapeDtypeStruct((E, OUT), jnp.float32),
    mesh=_MESH,
    scratch_types=[
        pltpu.VMEM((W_CHUNKS + 1, CH), jnp.int32),
        pltpu.VMEM((W_CHUNKS + 1, CH), jnp.int32),
        pltpu.VMEM((2, SS * CH, DP), jnp.float32),
        pltpu.VMEM((2, SS * CH, DP), jnp.float32),
        pltpu.SemaphoreType.DMA,
        pltpu.SemaphoreType.DMA,
        pltpu.SemaphoreType.DMA,
        pltpu.SemaphoreType.DMA,
    ],
    compiler_params=pltpu.CompilerParams(use_tc_tiling_on_sc=False),
)
def _gather_sc(h_hbm, eis_hbm, eid_hbm, g_hbm,
               idxs_v, idxd_v, rows_s, rows_d, sem_g, sem_g2, sem_ws, sem_wd):
    c = lax.axis_index("c")
    s = lax.axis_index("s")
    wid = s * NC + c
    base_ch = wid * W_CHUNKS

    pltpu.sync_copy(eis_hbm.at[pl.ds(base_ch, W_CHUNKS)],
                    idxs_v.at[pl.ds(0, W_CHUNKS)])
    pltpu.sync_copy(eid_hbm.at[pl.ds(base_ch, W_CHUNKS)],
                    idxd_v.at[pl.ds(0, W_CHUNKS)])

    @pl.when(wid < NCHUNK - NW * W_CHUNKS)
    def _():
        extra = NW * W_CHUNKS + wid
        pltpu.sync_copy(eis_hbm.at[pl.ds(extra, 1)],
                        idxs_v.at[pl.ds(W_CHUNKS, 1)])
        pltpu.sync_copy(eid_hbm.at[pl.ds(extra, 1)],
                        idxd_v.at[pl.ds(W_CHUNKS, 1)])

    pending_wb = {}

    def run_superstep(i, local_ch0, n, glob_ch0):
        b = i % 2
        if i >= 2:
            for d in pending_wb.pop(b):
                d.wait()
        gs, gd = [], []
        for j in range(n):
            gs.append(pltpu.async_copy(
                h_hbm.at[idxs_v.at[local_ch0 + j]],
                rows_s.at[b, pl.ds(j * CH, CH)], sem_g))
            gd.append(pltpu.async_copy(
                h_hbm.at[idxd_v.at[local_ch0 + j]],
                rows_d.at[b, pl.ds(j * CH, CH)], sem_g2))
        for d in gs + gd:
            d.wait()
        off = glob_ch0 * CH
        ws = pltpu.async_copy(
            rows_s.at[b, pl.ds(0, n * CH)],
            g_hbm.at[pl.ds(off, n * CH), pl.ds(0, DP)], sem_ws)
        wd = pltpu.async_copy(
            rows_d.at[b, pl.ds(0, n * CH)],
            g_hbm.at[pl.ds(off, n * CH), pl.ds(DP, DP)], sem_wd)
        pending_wb[b] = [ws, wd]

    for i, (k, n) in enumerate(_SUPERSTEPS):
        run_superstep(i, k, n, base_ch + k)

    ntail = len(_SUPERSTEPS)
    bt = ntail % 2
    if bt in pending_wb:
        for d in pending_wb.pop(bt):
            d.wait()

    @pl.when(wid < NCHUNK - NW * W_CHUNKS)
    def _():
        extra = NW * W_CHUNKS + wid
        g1 = pltpu.async_copy(h_hbm.at[idxs_v.at[W_CHUNKS]],
                              rows_s.at[bt, pl.ds(0, CH)], sem_g)
        g2 = pltpu.async_copy(h_hbm.at[idxd_v.at[W_CHUNKS]],
                              rows_d.at[bt, pl.ds(0, CH)], sem_g2)
        g1.wait()
        g2.wait()
        pltpu.sync_copy(rows_s.at[bt, pl.ds(0, CH)],
                        g_hbm.at[pl.ds(extra * CH, CH), pl.ds(0, DP)])
        pltpu.sync_copy(rows_d.at[bt, pl.ds(0, CH)],
                        g_hbm.at[pl.ds(extra * CH, CH), pl.ds(DP, DP)])

    for b in list(pending_wb):
        for d in pending_wb.pop(b):
            d.wait()


def _edge_body(g_ref, r_ref, s_ref, w2_ref, b2_ref, g2_ref,
               be2_ref, o_ref):
    g = g_ref[...].astype(jnp.bfloat16)
    a = jnp.dot(g[:, :DP], r_ref[...], preferred_element_type=jnp.float32)
    b = jnp.dot(g[:, DP:2 * DP], s_ref[...],
                preferred_element_type=jnp.float32)
    kron = (a * b).astype(jnp.bfloat16)
    y = jnp.dot(kron, w2_ref[...], preferred_element_type=jnp.float32)
    y = y + b2_ref[...]
    mu = jnp.mean(y, axis=1, keepdims=True)
    var = jnp.mean(y * y, axis=1, keepdims=True) - mu * mu
    h = (y - mu) * lax.rsqrt(var + 1e-5) * g2_ref[...] + be2_ref[...]
    o_ref[...] = jnp.maximum(h, 0.0)


def _edge_mlp(g, rmat, smat, w2, b2, g2, be2):
    blk = 1280
    grid = E // blk
    return pl.pallas_call(
        _edge_body,
        grid=(grid,),
        in_specs=[
            pl.BlockSpec((blk, OUT), lambda i: (i, 0)),
            pl.BlockSpec((DP, KRON), lambda i: (0, 0)),
            pl.BlockSpec((DP, KRON), lambda i: (0, 0)),
            pl.BlockSpec((KRON, OUT), lambda i: (0, 0)),
            pl.BlockSpec((1, OUT), lambda i: (0, 0)),
            pl.BlockSpec((1, OUT), lambda i: (0, 0)),
            pl.BlockSpec((1, OUT), lambda i: (0, 0)),
        ],
        out_specs=pl.BlockSpec((blk, OUT), lambda i: (i, 0)),
        out_shape=jax.ShapeDtypeStruct((E, OUT), jnp.float32),
    )(g, rmat, smat, w2, b2, g2, be2)


T_CHUNKS = CHUNKS_PER_CORE // NS


@functools.partial(
    pl.kernel,
    out_type=(
        jax.ShapeDtypeStruct((NP, OUT), jnp.float32),
        jax.ShapeDtypeStruct((NP, OUT), jnp.float32),
    ),
    mesh=_MESH,
    scratch_types=[
        pltpu.VMEM_SHARED((NP, OUT), jnp.float32),
        pltpu.VMEM((2, CH, OUT), jnp.float32),
        pltpu.VMEM((T_CHUNKS + 1, CH), jnp.int32),
        pltpu.SemaphoreType.DMA,
        pltpu.SemaphoreType.DMA,
    ],
    compiler_params=pltpu.CompilerParams(use_tc_tiling_on_sc=False),
)
def _scatter_sc(m_hbm, eid_hbm, zrows_hbm, p0_hbm, p1_hbm,
                acc, mv, idxv, sem_m0, sem_m1):
    c = lax.axis_index("c")
    s = lax.axis_index("s")
    base_ch = c * CHUNKS_PER_CORE + s * T_CHUNKS
    sems = (sem_m0, sem_m1)

    pltpu.sync_copy(eid_hbm.at[pl.ds(base_ch, T_CHUNKS)],
                    idxv.at[pl.ds(0, T_CHUNKS)])

    @pl.when(s == 0)
    def _():
        pltpu.sync_copy(eid_hbm.at[pl.ds(c * CHUNKS_PER_CORE + NS * T_CHUNKS, 1)],
                        idxv.at[pl.ds(T_CHUNKS, 1)])

    pltpu.sync_copy(zrows_hbm, mv.at[0])
    for j in range(ROWS_PER_TILE // CH):
        pltpu.sync_copy(mv.at[0], acc.at[pl.ds(s * ROWS_PER_TILE + j * CH, CH)])
    plsc.subcore_barrier()

    first = pltpu.async_copy(m_hbm.at[pl.ds(base_ch * CH, CH)], mv.at[0],
                             sem_m0)
    pending = {0: first}
    for i in range(T_CHUNKS):
        b = i % 2
        if i + 1 < T_CHUNKS:
            pending[1 - b] = pltpu.async_copy(
                m_hbm.at[pl.ds((base_ch + i + 1) * CH, CH)], mv.at[1 - b],
                sems[1 - b])
        pending.pop(b).wait()
        pltpu.sync_copy(mv.at[b], acc.at[idxv.at[i]], add=True)

    @pl.when(s == 0)
    def _():
        extra_ch = c * CHUNKS_PER_CORE + NS * T_CHUNKS
        pltpu.sync_copy(m_hbm.at[pl.ds(extra_ch * CH, CH)], mv.at[0])
        pltpu.sync_copy(mv.at[0], acc.at[idxv.at[T_CHUNKS]], add=True)

    plsc.subcore_barrier()
    for j in range(ROWS_PER_TILE // CH):
        row = s * ROWS_PER_TILE + j * CH
        pltpu.sync_copy(acc.at[pl.ds(row, CH)], mv.at[j % 2])

        @pl.when(c == 0)
        def _():
            pltpu.sync_copy(mv.at[j % 2], p0_hbm.at[pl.ds(row, CH)])

        @pl.when(c == 1)
        def _():
            pltpu.sync_copy(mv.at[j % 2], p1_hbm.at[pl.ds(row, CH)])


def _combine_body(p0_ref, p1_ref, o_ref):
    o_ref[...] = p0_ref[...] + p1_ref[...]


def _combine(p0, p1):
    blk = 2000
    grid = N // blk
    return pl.pallas_call(
        _combine_body,
        grid=(grid,),
        in_specs=[
            pl.BlockSpec((blk, OUT), lambda i: (i, 0)),
            pl.BlockSpec((blk, OUT), lambda i: (i, 0)),
        ],
        out_specs=pl.BlockSpec((blk, OUT), lambda i: (i, 0)),
        out_shape=jax.ShapeDtypeStruct((N, OUT), jnp.float32),
    )(p0, p1)


def _build_rs():
    r = np.zeros((DP, KRON), np.float32)
    s = np.zeros((DP, KRON), np.float32)
    for a in range(20):
        for k in range(20):
            r[a, a * 20 + k] = 1.0
            s[k, a * 20 + k] = 1.0
    return r, s


_R_NP, _S_NP = _build_rs()


def kernel(node_feat, edge_index, W1, b1, g1, be1, W2, b2, g2, be2):
    w1p = jnp.pad(W1, ((0, 0), (0, DP - 20)))
    b1p = jnp.pad(b1, (0, DP - 20)).reshape(1, DP)
    g1p = jnp.pad(g1, (0, DP - 20)).reshape(1, DP)
    be1p = jnp.pad(be1, (0, DP - 20)).reshape(1, DP)
    ei_src = edge_index[0].reshape(NCHUNK, CH)
    ei_dst = edge_index[1].reshape(NCHUNK, CH)
    zrows = jnp.zeros((CH, OUT), jnp.float32)

    h32 = _node_proj(node_feat, w1p, b1p, g1p, be1p)
    g = _gather_sc(h32, ei_src, ei_dst)
    m = _edge_mlp(g,
                  jnp.asarray(_R_NP, dtype=jnp.bfloat16),
                  jnp.asarray(_S_NP, dtype=jnp.bfloat16),
                  W2.astype(jnp.bfloat16),
                  b2.reshape(1, OUT), g2.reshape(1, OUT), be2.reshape(1, OUT))
    p0, p1 = _scatter_sc(m, ei_dst, zrows)
    return _combine(p0, p1)


# speedup vs baseline: 1.9727x; 1.1245x over previous
import functools

import jax
import jax.numpy as jnp
import numpy as np
from jax import lax
from jax.experimental import pallas as pl
from jax.experimental.pallas import tpu as pltpu
from jax.experimental.pallas import tpu_sc as plsc

N = 10000
E = 160000
D = 128
OUT = 128
DP = 32
KRON = 400

NC = 2
NS = 16
NW = NC * NS
CH = 128
NCHUNK = E // CH
CHUNKS_PER_CORE = NCHUNK // NC
NP = 10240
ROWS_PER_TILE = NP // NS


def _node_proj_body(x_ref, w_ref, b_ref, g_ref, be_ref, o_ref):
    y = jnp.dot(x_ref[...], w_ref[...], preferred_element_type=jnp.float32)
    y = y + b_ref[...]
    mu = jnp.sum(y, axis=1, keepdims=True) * (1.0 / 20.0)
    var = jnp.sum(y * y, axis=1, keepdims=True) * (1.0 / 20.0) - mu * mu
    h = (y - mu) * lax.rsqrt(var + 1e-5) * g_ref[...] + be_ref[...]
    o_ref[...] = jnp.maximum(h, 0.0)


def _node_proj(x, w1p, b1p, g1p, be1p):
    blk = 2000
    grid = N // blk
    return pl.pallas_call(
        _node_proj_body,
        grid=(grid,),
        in_specs=[
            pl.BlockSpec((blk, D), lambda i: (i, 0)),
            pl.BlockSpec((D, DP), lambda i: (0, 0)),
            pl.BlockSpec((1, DP), lambda i: (0, 0)),
            pl.BlockSpec((1, DP), lambda i: (0, 0)),
            pl.BlockSpec((1, DP), lambda i: (0, 0)),
        ],
        out_specs=pl.BlockSpec((blk, DP), lambda i: (i, 0)),
        out_shape=jax.ShapeDtypeStruct((N, DP), jnp.float32),
    )(x, w1p, b1p, g1p, be1p)


_MESH = plsc.VectorSubcoreMesh(
    core_axis_name="c", subcore_axis_name="s", num_cores=NC, num_subcores=NS)


W_CHUNKS = NCHUNK // NW
SS = 6
_SUPERSTEPS = [(k, min(SS, W_CHUNKS - k)) for k in range(0, W_CHUNKS, SS)]


@functools.partial(
    pl.kernel,
    out_type=jax.ShapeDtypeStruct((E, OUT), jnp.float32),
    mesh=_MESH,
    scratch_types=[
        pltpu.VMEM((W_CHUNKS + 1, CH), jnp.int32),
        pltpu.VMEM((W_CHUNKS + 1, CH), jnp.int32),
        pltpu.VMEM((2, SS * CH, DP), jnp.float32),
        pltpu.VMEM((2, SS * CH, DP), jnp.float32),
        pltpu.SemaphoreType.DMA,
        pltpu.SemaphoreType.DMA,
        pltpu.SemaphoreType.DMA,
        pltpu.SemaphoreType.DMA,
    ],
    compiler_params=pltpu.CompilerParams(use_tc_tiling_on_sc=False),
)
def _gather_sc(h_hbm, eis_hbm, eid_hbm, g_hbm,
               idxs_v, idxd_v, rows_s, rows_d, sem_g, sem_g2, sem_ws, sem_wd):
    c = lax.axis_index("c")
    s = lax.axis_index("s")
    wid = s * NC + c
    base_ch = wid * W_CHUNKS

    pltpu.sync_copy(eis_hbm.at[pl.ds(base_ch, W_CHUNKS)],
                    idxs_v.at[pl.ds(0, W_CHUNKS)])
    pltpu.sync_copy(eid_hbm.at[pl.ds(base_ch, W_CHUNKS)],
                    idxd_v.at[pl.ds(0, W_CHUNKS)])

    @pl.when(wid < NCHUNK - NW * W_CHUNKS)
    def _():
        extra = NW * W_CHUNKS + wid
        pltpu.sync_copy(eis_hbm.at[pl.ds(extra, 1)],
                        idxs_v.at[pl.ds(W_CHUNKS, 1)])
        pltpu.sync_copy(eid_hbm.at[pl.ds(extra, 1)],
                        idxd_v.at[pl.ds(W_CHUNKS, 1)])

    pending_wb = {}

    def run_superstep(i, local_ch0, n, glob_ch0):
        b = i % 2
        if i >= 2:
            for d in pending_wb.pop(b):
                d.wait()
        gs, gd = [], []
        for j in range(n):
            gs.append(pltpu.async_copy(
                h_hbm.at[idxs_v.at[local_ch0 + j]],
                rows_s.at[b, pl.ds(j * CH, CH)], sem_g))
            gd.append(pltpu.async_copy(
                h_hbm.at[idxd_v.at[local_ch0 + j]],
                rows_d.at[b, pl.ds(j * CH, CH)], sem_g2))
        for d in gs + gd:
            d.wait()
        off = glob_ch0 * CH
        ws = pltpu.async_copy(
            rows_s.at[b, pl.ds(0, n * CH)],
            g_hbm.at[pl.ds(off, n * CH), pl.ds(0, DP)], sem_ws)
        wd = pltpu.async_copy(
            rows_d.at[b, pl.ds(0, n * CH)],
            g_hbm.at[pl.ds(off, n * CH), pl.ds(DP, DP)], sem_wd)
        pending_wb[b] = [ws, wd]

    for i, (k, n) in enumerate(_SUPERSTEPS):
        run_superstep(i, k, n, base_ch + k)

    ntail = len(_SUPERSTEPS)
    bt = ntail % 2
    if bt in pending_wb:
        for d in pending_wb.pop(bt):
            d.wait()

    @pl.when(wid < NCHUNK - NW * W_CHUNKS)
    def _():
        extra = NW * W_CHUNKS + wid
        g1 = pltpu.async_copy(h_hbm.at[idxs_v.at[W_CHUNKS]],
                              rows_s.at[bt, pl.ds(0, CH)], sem_g)
        g2 = pltpu.async_copy(h_hbm.at[idxd_v.at[W_CHUNKS]],
                              rows_d.at[bt, pl.ds(0, CH)], sem_g2)
        g1.wait()
        g2.wait()
        pltpu.sync_copy(rows_s.at[bt, pl.ds(0, CH)],
                        g_hbm.at[pl.ds(extra * CH, CH), pl.ds(0, DP)])
        pltpu.sync_copy(rows_d.at[bt, pl.ds(0, CH)],
                        g_hbm.at[pl.ds(extra * CH, CH), pl.ds(DP, DP)])

    for b in list(pending_wb):
        for d in pending_wb.pop(b):
            d.wait()


def _edge_body(g_ref, r_ref, s_ref, w2_ref, b2_ref, g2_ref,
               be2_ref, o_ref):
    g = g_ref[...].astype(jnp.bfloat16)
    a = jnp.dot(g[:, :DP], r_ref[...], preferred_element_type=jnp.float32)
    b = jnp.dot(g[:, DP:2 * DP], s_ref[...],
                preferred_element_type=jnp.float32)
    kron = (a * b).astype(jnp.bfloat16)
    y = jnp.dot(kron, w2_ref[...], preferred_element_type=jnp.float32)
    y = y + b2_ref[...]
    mu = jnp.mean(y, axis=1, keepdims=True)
    var = jnp.mean(y * y, axis=1, keepdims=True) - mu * mu
    h = (y - mu) * lax.rsqrt(var + 1e-5) * g2_ref[...] + be2_ref[...]
    o_ref[...] = jnp.maximum(h, 0.0)


def _edge_mlp(g, rmat, smat, w2, b2, g2, be2):
    blk = 6400
    grid = E // blk
    return pl.pallas_call(
        _edge_body,
        grid=(grid,),
        in_specs=[
            pl.BlockSpec((blk, OUT), lambda i: (i, 0)),
            pl.BlockSpec((DP, KRON), lambda i: (0, 0)),
            pl.BlockSpec((DP, KRON), lambda i: (0, 0)),
            pl.BlockSpec((KRON, OUT), lambda i: (0, 0)),
            pl.BlockSpec((1, OUT), lambda i: (0, 0)),
            pl.BlockSpec((1, OUT), lambda i: (0, 0)),
            pl.BlockSpec((1, OUT), lambda i: (0, 0)),
        ],
        out_specs=pl.BlockSpec((blk, OUT), lambda i: (i, 0)),
        out_shape=jax.ShapeDtypeStruct((E, OUT), jnp.float32),
    )(g, rmat, smat, w2, b2, g2, be2)


T_CHUNKS = CHUNKS_PER_CORE // NS


@functools.partial(
    pl.kernel,
    out_type=(
        jax.ShapeDtypeStruct((NP, OUT), jnp.float32),
        jax.ShapeDtypeStruct((NP, OUT), jnp.float32),
    ),
    mesh=_MESH,
    scratch_types=[
        pltpu.VMEM_SHARED((NP, OUT), jnp.float32),
        pltpu.VMEM((2, CH, OUT), jnp.float32),
        pltpu.VMEM((T_CHUNKS + 1, CH), jnp.int32),
        pltpu.SemaphoreType.DMA,
        pltpu.SemaphoreType.DMA,
    ],
    compiler_params=pltpu.CompilerParams(use_tc_tiling_on_sc=False),
)
def _scatter_sc(m_hbm, eid_hbm, zrows_hbm, p0_hbm, p1_hbm,
                acc, mv, idxv, sem_m0, sem_m1):
    c = lax.axis_index("c")
    s = lax.axis_index("s")
    base_ch = c * CHUNKS_PER_CORE + s * T_CHUNKS
    sems = (sem_m0, sem_m1)

    pltpu.sync_copy(eid_hbm.at[pl.ds(base_ch, T_CHUNKS)],
                    idxv.at[pl.ds(0, T_CHUNKS)])

    @pl.when(s == 0)
    def _():
        pltpu.sync_copy(eid_hbm.at[pl.ds(c * CHUNKS_PER_CORE + NS * T_CHUNKS, 1)],
                        idxv.at[pl.ds(T_CHUNKS, 1)])

    pltpu.sync_copy(zrows_hbm, mv.at[0])
    for j in range(ROWS_PER_TILE // CH):
        pltpu.sync_copy(mv.at[0], acc.at[pl.ds(s * ROWS_PER_TILE + j * CH, CH)])
    plsc.subcore_barrier()

    first = pltpu.async_copy(m_hbm.at[pl.ds(base_ch * CH, CH)], mv.at[0],
                             sem_m0)
    pending = {0: first}
    for i in range(T_CHUNKS):
        b = i % 2
        if i + 1 < T_CHUNKS:
            pending[1 - b] = pltpu.async_copy(
                m_hbm.at[pl.ds((base_ch + i + 1) * CH, CH)], mv.at[1 - b],
                sems[1 - b])
        pending.pop(b).wait()
        pltpu.sync_copy(mv.at[b], acc.at[idxv.at[i]], add=True)

    @pl.when(s == 0)
    def _():
        extra_ch = c * CHUNKS_PER_CORE + NS * T_CHUNKS
        pltpu.sync_copy(m_hbm.at[pl.ds(extra_ch * CH, CH)], mv.at[0])
        pltpu.sync_copy(mv.at[0], acc.at[idxv.at[T_CHUNKS]], add=True)

    plsc.subcore_barrier()
    for j in range(ROWS_PER_TILE // CH):
        row = s * ROWS_PER_TILE + j * CH
        pltpu.sync_copy(acc.at[pl.ds(row, CH)], mv.at[j % 2])

        @pl.when(c == 0)
        def _():
            pltpu.sync_copy(mv.at[j % 2], p0_hbm.at[pl.ds(row, CH)])

        @pl.when(c == 1)
        def _():
            pltpu.sync_copy(mv.at[j % 2], p1_hbm.at[pl.ds(row, CH)])


def _combine_body(p0_ref, p1_ref, o_ref):
    o_ref[...] = p0_ref[...] + p1_ref[...]


def _combine(p0, p1):
    blk = 2000
    grid = N // blk
    return pl.pallas_call(
        _combine_body,
        grid=(grid,),
        in_specs=[
            pl.BlockSpec((blk, OUT), lambda i: (i, 0)),
            pl.BlockSpec((blk, OUT), lambda i: (i, 0)),
        ],
        out_specs=pl.BlockSpec((blk, OUT), lambda i: (i, 0)),
        out_shape=jax.ShapeDtypeStruct((N, OUT), jnp.float32),
    )(p0, p1)


def _build_rs():
    r = np.zeros((DP, KRON), np.float32)
    s = np.zeros((DP, KRON), np.float32)
    for a in range(20):
        for k in range(20):
            r[a, a * 20 + k] = 1.0
            s[k, a * 20 + k] = 1.0
    return r, s


_R_NP, _S_NP = _build_rs()


def kernel(node_feat, edge_index, W1, b1, g1, be1, W2, b2, g2, be2):
    w1p = jnp.pad(W1, ((0, 0), (0, DP - 20)))
    b1p = jnp.pad(b1, (0, DP - 20)).reshape(1, DP)
    g1p = jnp.pad(g1, (0, DP - 20)).reshape(1, DP)
    be1p = jnp.pad(be1, (0, DP - 20)).reshape(1, DP)
    ei_src = edge_index[0].reshape(NCHUNK, CH)
    ei_dst = edge_index[1].reshape(NCHUNK, CH)
    zrows = jnp.zeros((CH, OUT), jnp.float32)

    h32 = _node_proj(node_feat, w1p, b1p, g1p, be1p)
    g = _gather_sc(h32, ei_src, ei_dst)
    m = _edge_mlp(g,
                  jnp.asarray(_R_NP, dtype=jnp.bfloat16),
                  jnp.asarray(_S_NP, dtype=jnp.bfloat16),
                  W2.astype(jnp.bfloat16),
                  b2.reshape(1, OUT), g2.reshape(1, OUT), be2.reshape(1, OUT))
    p0, p1 = _scatter_sc(m, ei_dst, zrows)
    return _combine(p0, p1)


# speedup vs baseline: 2.1893x; 1.1098x over previous
import functools

import jax
import jax.numpy as jnp
import numpy as np
from jax import lax
from jax.experimental import pallas as pl
from jax.experimental.pallas import tpu as pltpu
from jax.experimental.pallas import tpu_sc as plsc

N = 10000
E = 160000
D = 128
OUT = 128
DP = 32
KRON = 400

NC = 2
NS = 16
NW = NC * NS
CH = 128
NCHUNK = E // CH
NP = 10240
ROWS_PER_TILE = NP // NS

HALF0 = 640
HALF1 = NCHUNK - HALF0


def _node_proj_body(x_ref, w_ref, b_ref, g_ref, be_ref, o_ref):
    y = jnp.dot(x_ref[...], w_ref[...], preferred_element_type=jnp.float32)
    y = y + b_ref[...]
    mu = jnp.sum(y, axis=1, keepdims=True) * (1.0 / 20.0)
    var = jnp.sum(y * y, axis=1, keepdims=True) * (1.0 / 20.0) - mu * mu
    h = (y - mu) * lax.rsqrt(var + 1e-5) * g_ref[...] + be_ref[...]
    o_ref[...] = jnp.maximum(h, 0.0)


def _node_proj(x, w1p, b1p, g1p, be1p):
    blk = 2000
    grid = N // blk
    return pl.pallas_call(
        _node_proj_body,
        grid=(grid,),
        in_specs=[
            pl.BlockSpec((blk, D), lambda i: (i, 0)),
            pl.BlockSpec((D, DP), lambda i: (0, 0)),
            pl.BlockSpec((1, DP), lambda i: (0, 0)),
            pl.BlockSpec((1, DP), lambda i: (0, 0)),
            pl.BlockSpec((1, DP), lambda i: (0, 0)),
        ],
        out_specs=pl.BlockSpec((blk, DP), lambda i: (i, 0)),
        out_shape=jax.ShapeDtypeStruct((N, DP), jnp.float32),
    )(x, w1p, b1p, g1p, be1p)


_MESH = plsc.VectorSubcoreMesh(
    core_axis_name="c", subcore_axis_name="s", num_cores=NC, num_subcores=NS)


def _make_gather(base_ch, count):
    wch = count // NW
    n_extra = count - NW * wch
    ss = 5
    supersteps = [(k, min(ss, wch - k)) for k in range(0, wch, ss)]
    e_out = count * CH

    @functools.partial(
        pl.kernel,
        out_type=jax.ShapeDtypeStruct((e_out, OUT), jnp.float32),
        mesh=_MESH,
        scratch_types=[
            pltpu.VMEM((wch + 1, CH), jnp.int32),
            pltpu.VMEM((wch + 1, CH), jnp.int32),
            pltpu.VMEM((2, ss * CH, DP), jnp.float32),
            pltpu.VMEM((2, ss * CH, DP), jnp.float32),
            pltpu.SemaphoreType.DMA,
            pltpu.SemaphoreType.DMA,
            pltpu.SemaphoreType.DMA,
            pltpu.SemaphoreType.DMA,
        ],
        compiler_params=pltpu.CompilerParams(use_tc_tiling_on_sc=False),
    )
    def gather(h_hbm, eis_hbm, eid_hbm, g_hbm,
               idxs_v, idxd_v, rows_s, rows_d, sem_g, sem_g2, sem_ws, sem_wd):
        c = lax.axis_index("c")
        s = lax.axis_index("s")
        wid = s * NC + c
        wbase = base_ch + wid * wch

        pltpu.sync_copy(eis_hbm.at[pl.ds(wbase, wch)],
                        idxs_v.at[pl.ds(0, wch)])
        pltpu.sync_copy(eid_hbm.at[pl.ds(wbase, wch)],
                        idxd_v.at[pl.ds(0, wch)])
        if n_extra:
            @pl.when(wid < n_extra)
            def _():
                extra = base_ch + NW * wch + wid
                pltpu.sync_copy(eis_hbm.at[pl.ds(extra, 1)],
                                idxs_v.at[pl.ds(wch, 1)])
                pltpu.sync_copy(eid_hbm.at[pl.ds(extra, 1)],
                                idxd_v.at[pl.ds(wch, 1)])

        pending_wb = {}

        def run_superstep(i, k, n):
            b = i % 2
            if i >= 2:
                for d in pending_wb.pop(b):
                    d.wait()
            descs = []
            for j in range(n):
                descs.append(pltpu.async_copy(
                    h_hbm.at[idxs_v.at[k + j]],
                    rows_s.at[b, pl.ds(j * CH, CH)], sem_g))
                descs.append(pltpu.async_copy(
                    h_hbm.at[idxd_v.at[k + j]],
                    rows_d.at[b, pl.ds(j * CH, CH)], sem_g2))
            for d in descs:
                d.wait()
            off = (wid * wch + k) * CH
            ws = pltpu.async_copy(
                rows_s.at[b, pl.ds(0, n * CH)],
                g_hbm.at[pl.ds(off, n * CH), pl.ds(0, DP)], sem_ws)
            wd = pltpu.async_copy(
                rows_d.at[b, pl.ds(0, n * CH)],
                g_hbm.at[pl.ds(off, n * CH), pl.ds(DP, DP)], sem_wd)
            pending_wb[b] = [ws, wd]

        for i, (k, n) in enumerate(supersteps):
            run_superstep(i, k, n)

        bt = len(supersteps) % 2
        if bt in pending_wb:
            for d in pending_wb.pop(bt):
                d.wait()

        if n_extra:
            @pl.when(wid < n_extra)
            def _():
                loc = (NW * wch + wid) * CH
                g1 = pltpu.async_copy(h_hbm.at[idxs_v.at[wch]],
                                      rows_s.at[bt, pl.ds(0, CH)], sem_g)
                g2 = pltpu.async_copy(h_hbm.at[idxd_v.at[wch]],
                                      rows_d.at[bt, pl.ds(0, CH)], sem_g2)
                g1.wait()
                g2.wait()
                pltpu.sync_copy(rows_s.at[bt, pl.ds(0, CH)],
                                g_hbm.at[pl.ds(loc, CH), pl.ds(0, DP)])
                pltpu.sync_copy(rows_d.at[bt, pl.ds(0, CH)],
                                g_hbm.at[pl.ds(loc, CH), pl.ds(DP, DP)])

        for b in list(pending_wb):
            for d in pending_wb.pop(b):
                d.wait()

    return gather


_GATHER0 = _make_gather(0, HALF0)
_GATHER1 = _make_gather(HALF0, HALF1)


def _edge_body(g_ref, r_ref, s_ref, w2_ref, b2_ref, g2_ref,
               be2_ref, o_ref):
    g = g_ref[...].astype(jnp.bfloat16)
    a = jnp.dot(g[:, :DP], r_ref[...], preferred_element_type=jnp.float32)
    b = jnp.dot(g[:, DP:2 * DP], s_ref[...],
                preferred_element_type=jnp.float32)
    kron = (a * b).astype(jnp.bfloat16)
    y = jnp.dot(kron, w2_ref[...], preferred_element_type=jnp.float32)
    y = y + b2_ref[...]
    mu = jnp.mean(y, axis=1, keepdims=True)
    var = jnp.mean(y * y, axis=1, keepdims=True) - mu * mu
    h = (y - mu) * lax.rsqrt(var + 1e-5) * g2_ref[...] + be2_ref[...]
    o_ref[...] = jnp.maximum(h, 0.0)


def _edge_mlp(g, rmat, smat, w2, b2, g2, be2, blk):
    e = g.shape[0]
    grid = e // blk
    return pl.pallas_call(
        _edge_body,
        grid=(grid,),
        in_specs=[
            pl.BlockSpec((blk, OUT), lambda i: (i, 0)),
            pl.BlockSpec((DP, KRON), lambda i: (0, 0)),
            pl.BlockSpec((DP, KRON), lambda i: (0, 0)),
            pl.BlockSpec((KRON, OUT), lambda i: (0, 0)),
            pl.BlockSpec((1, OUT), lambda i: (0, 0)),
            pl.BlockSpec((1, OUT), lambda i: (0, 0)),
            pl.BlockSpec((1, OUT), lambda i: (0, 0)),
        ],
        out_specs=pl.BlockSpec((blk, OUT), lambda i: (i, 0)),
        out_shape=jax.ShapeDtypeStruct((e, OUT), jnp.float32),
    )(g, rmat, smat, w2, b2, g2, be2)


def _make_scatter(base_ch, count):
    kc = count // NC
    tc = kc // NS
    n_extra_t = kc - NS * tc

    @functools.partial(
        pl.kernel,
        out_type=(
            jax.ShapeDtypeStruct((NP, OUT), jnp.float32),
            jax.ShapeDtypeStruct((NP, OUT), jnp.float32),
        ),
        mesh=_MESH,
        scratch_types=[
            pltpu.VMEM_SHARED((NP, OUT), jnp.float32),
            pltpu.VMEM((2, CH, OUT), jnp.float32),
            pltpu.VMEM((tc + 1, CH), jnp.int32),
            pltpu.SemaphoreType.DMA,
            pltpu.SemaphoreType.DMA,
        ],
        compiler_params=pltpu.CompilerParams(use_tc_tiling_on_sc=False),
    )
    def scatter(m_hbm, eid_hbm, zrows_hbm, p0_hbm, p1_hbm,
                acc, mv, idxv, sem_m0, sem_m1):
        c = lax.axis_index("c")
        s = lax.axis_index("s")
        loc0 = c * kc + s * tc
        sems = (sem_m0, sem_m1)

        pltpu.sync_copy(eid_hbm.at[pl.ds(base_ch + loc0, tc)],
                        idxv.at[pl.ds(0, tc)])
        if n_extra_t:
            @pl.when(s < n_extra_t)
            def _():
                ex = c * kc + NS * tc + s
                pltpu.sync_copy(eid_hbm.at[pl.ds(base_ch + ex, 1)],
                                idxv.at[pl.ds(tc, 1)])

        pltpu.sync_copy(zrows_hbm, mv.at[0])
        for j in range(ROWS_PER_TILE // CH):
            pltpu.sync_copy(mv.at[0],
                            acc.at[pl.ds(s * ROWS_PER_TILE + j * CH, CH)])
        plsc.subcore_barrier()

        pending = {0: pltpu.async_copy(
            m_hbm.at[pl.ds(loc0 * CH, CH)], mv.at[0], sem_m0)}
        for i in range(tc):
            b = i % 2
            if i + 1 < tc:
                pending[1 - b] = pltpu.async_copy(
                    m_hbm.at[pl.ds((loc0 + i + 1) * CH, CH)], mv.at[1 - b],
                    sems[1 - b])
            pending.pop(b).wait()
            pltpu.sync_copy(mv.at[b], acc.at[idxv.at[i]], add=True)

        if n_extra_t:
            @pl.when(s < n_extra_t)
            def _():
                ex = c * kc + NS * tc + s
                pltpu.sync_copy(m_hbm.at[pl.ds(ex * CH, CH)], mv.at[0])
                pltpu.sync_copy(mv.at[0], acc.at[idxv.at[tc]], add=True)

        plsc.subcore_barrier()
        for j in range(ROWS_PER_TILE // CH):
            row = s * ROWS_PER_TILE + j * CH
            pltpu.sync_copy(acc.at[pl.ds(row, CH)], mv.at[j % 2])

            @pl.when(c == 0)
            def _():
                pltpu.sync_copy(mv.at[j % 2], p0_hbm.at[pl.ds(row, CH)])

            @pl.when(c == 1)
            def _():
                pltpu.sync_copy(mv.at[j % 2], p1_hbm.at[pl.ds(row, CH)])

    return scatter


_SCATTER0 = _make_scatter(0, HALF0)
_SCATTER1 = _make_scatter(HALF0, HALF1)


def _combine_body(p0_ref, p1_ref, p2_ref, p3_ref, o_ref):
    o_ref[...] = (p0_ref[...] + p1_ref[...]) + (p2_ref[...] + p3_ref[...])


def _combine(p0, p1, p2, p3):
    blk = 2000
    grid = N // blk
    spec = pl.BlockSpec((blk, OUT), lambda i: (i, 0))
    return pl.pallas_call(
        _combine_body,
        grid=(grid,),
        in_specs=[spec, spec, spec, spec],
        out_specs=spec,
        out_shape=jax.ShapeDtypeStruct((N, OUT), jnp.float32),
    )(p0, p1, p2, p3)


def _build_rs():
    r = np.zeros((DP, KRON), np.float32)
    s = np.zeros((DP, KRON), np.float32)
    for a in range(20):
        for k in range(20):
            r[a, a * 20 + k] = 1.0
            s[k, a * 20 + k] = 1.0
    return r, s


_R_NP, _S_NP = _build_rs()


def kernel(node_feat, edge_index, W1, b1, g1, be1, W2, b2, g2, be2):
    w1p = jnp.pad(W1, ((0, 0), (0, DP - 20)))
    b1p = jnp.pad(b1, (0, DP - 20)).reshape(1, DP)
    g1p = jnp.pad(g1, (0, DP - 20)).reshape(1, DP)
    be1p = jnp.pad(be1, (0, DP - 20)).reshape(1, DP)
    ei_src = edge_index[0].reshape(NCHUNK, CH)
    ei_dst = edge_index[1].reshape(NCHUNK, CH)
    zrows = jnp.zeros((CH, OUT), jnp.float32)
    rmat = jnp.asarray(_R_NP, dtype=jnp.bfloat16)
    smat = jnp.asarray(_S_NP, dtype=jnp.bfloat16)
    w2b = W2.astype(jnp.bfloat16)
    b2r = b2.reshape(1, OUT)
    g2r = g2.reshape(1, OUT)
    be2r = be2.reshape(1, OUT)

    h32 = _node_proj(node_feat, w1p, b1p, g1p, be1p)
    g0 = _GATHER0(h32, ei_src, ei_dst)
    g1_ = _GATHER1(h32, ei_src, ei_dst)
    m0 = _edge_mlp(g0, rmat, smat, w2b, b2r, g2r, be2r, blk=5120)
    m1 = _edge_mlp(g1_, rmat, smat, w2b, b2r, g2r, be2r, blk=4880)
    pa0, pa1 = _SCATTER0(m0, ei_dst, zrows)
    pb0, pb1 = _SCATTER1(m1, ei_dst, zrows)
    return _combine(pa0, pa1, pb0, pb1)


# speedup vs baseline: 2.2311x; 1.0191x over previous
import functools

import jax
import jax.numpy as jnp
import numpy as np
from jax import lax
from jax.experimental import pallas as pl
from jax.experimental.pallas import tpu as pltpu
from jax.experimental.pallas import tpu_sc as plsc

N = 10000
E = 160000
D = 128
OUT = 128
DP = 32
KRON = 400

NC = 2
NS = 16
NW = NC * NS
CH = 128
NCHUNK = E // CH
NP = 10240
ROWS_PER_TILE = NP // NS

PARTS = [320, 320, 320, 290]
assert sum(PARTS) == NCHUNK


def _node_proj_body(x_ref, w_ref, b_ref, g_ref, be_ref, o_ref):
    y = jnp.dot(x_ref[...], w_ref[...], preferred_element_type=jnp.float32)
    y = y + b_ref[...]
    mu = jnp.sum(y, axis=1, keepdims=True) * (1.0 / 20.0)
    var = jnp.sum(y * y, axis=1, keepdims=True) * (1.0 / 20.0) - mu * mu
    h = (y - mu) * lax.rsqrt(var + 1e-5) * g_ref[...] + be_ref[...]
    o_ref[...] = jnp.maximum(h, 0.0)


def _node_proj(x, w1p, b1p, g1p, be1p):
    blk = 2000
    grid = N // blk
    return pl.pallas_call(
        _node_proj_body,
        grid=(grid,),
        in_specs=[
            pl.BlockSpec((blk, D), lambda i: (i, 0)),
            pl.BlockSpec((D, DP), lambda i: (0, 0)),
            pl.BlockSpec((1, DP), lambda i: (0, 0)),
            pl.BlockSpec((1, DP), lambda i: (0, 0)),
            pl.BlockSpec((1, DP), lambda i: (0, 0)),
        ],
        out_specs=pl.BlockSpec((blk, DP), lambda i: (i, 0)),
        out_shape=jax.ShapeDtypeStruct((N, DP), jnp.float32),
    )(x, w1p, b1p, g1p, be1p)


_MESH = plsc.VectorSubcoreMesh(
    core_axis_name="c", subcore_axis_name="s", num_cores=NC, num_subcores=NS)


def _make_gather(base_ch, count):
    wch = count // NW
    n_extra = count - NW * wch
    ss = 5
    supersteps = [(k, min(ss, wch - k)) for k in range(0, wch, ss)]
    e_out = count * CH

    @functools.partial(
        pl.kernel,
        out_type=jax.ShapeDtypeStruct((e_out, OUT), jnp.float32),
        mesh=_MESH,
        scratch_types=[
            pltpu.VMEM((wch + 1, CH), jnp.int32),
            pltpu.VMEM((wch + 1, CH), jnp.int32),
            pltpu.VMEM((2, ss * CH, DP), jnp.float32),
            pltpu.VMEM((2, ss * CH, DP), jnp.float32),
            pltpu.SemaphoreType.DMA,
            pltpu.SemaphoreType.DMA,
            pltpu.SemaphoreType.DMA,
            pltpu.SemaphoreType.DMA,
        ],
        compiler_params=pltpu.CompilerParams(use_tc_tiling_on_sc=False),
    )
    def gather(h_hbm, eis_hbm, eid_hbm, g_hbm,
               idxs_v, idxd_v, rows_s, rows_d, sem_g, sem_g2, sem_ws, sem_wd):
        c = lax.axis_index("c")
        s = lax.axis_index("s")
        wid = s * NC + c
        wbase = base_ch + wid * wch

        pltpu.sync_copy(eis_hbm.at[pl.ds(wbase, wch)],
                        idxs_v.at[pl.ds(0, wch)])
        pltpu.sync_copy(eid_hbm.at[pl.ds(wbase, wch)],
                        idxd_v.at[pl.ds(0, wch)])
        if n_extra:
            @pl.when(wid < n_extra)
            def _():
                extra = base_ch + NW * wch + wid
                pltpu.sync_copy(eis_hbm.at[pl.ds(extra, 1)],
                                idxs_v.at[pl.ds(wch, 1)])
                pltpu.sync_copy(eid_hbm.at[pl.ds(extra, 1)],
                                idxd_v.at[pl.ds(wch, 1)])

        pending_wb = {}

        def run_superstep(i, k, n):
            b = i % 2
            if i >= 2:
                for d in pending_wb.pop(b):
                    d.wait()
            descs = []
            for j in range(n):
                descs.append(pltpu.async_copy(
                    h_hbm.at[idxs_v.at[k + j]],
                    rows_s.at[b, pl.ds(j * CH, CH)], sem_g))
                descs.append(pltpu.async_copy(
                    h_hbm.at[idxd_v.at[k + j]],
                    rows_d.at[b, pl.ds(j * CH, CH)], sem_g2))
            for d in descs:
                d.wait()
            off = (wid * wch + k) * CH
            ws = pltpu.async_copy(
                rows_s.at[b, pl.ds(0, n * CH)],
                g_hbm.at[pl.ds(off, n * CH), pl.ds(0, DP)], sem_ws)
            wd = pltpu.async_copy(
                rows_d.at[b, pl.ds(0, n * CH)],
                g_hbm.at[pl.ds(off, n * CH), pl.ds(DP, DP)], sem_wd)
            pending_wb[b] = [ws, wd]

        for i, (k, n) in enumerate(supersteps):
            run_superstep(i, k, n)

        bt = len(supersteps) % 2
        if bt in pending_wb:
            for d in pending_wb.pop(bt):
                d.wait()

        if n_extra:
            @pl.when(wid < n_extra)
            def _():
                loc = (NW * wch + wid) * CH
                g1 = pltpu.async_copy(h_hbm.at[idxs_v.at[wch]],
                                      rows_s.at[bt, pl.ds(0, CH)], sem_g)
                g2 = pltpu.async_copy(h_hbm.at[idxd_v.at[wch]],
                                      rows_d.at[bt, pl.ds(0, CH)], sem_g2)
                g1.wait()
                g2.wait()
                pltpu.sync_copy(rows_s.at[bt, pl.ds(0, CH)],
                                g_hbm.at[pl.ds(loc, CH), pl.ds(0, DP)])
                pltpu.sync_copy(rows_d.at[bt, pl.ds(0, CH)],
                                g_hbm.at[pl.ds(loc, CH), pl.ds(DP, DP)])

        for b in list(pending_wb):
            for d in pending_wb.pop(b):
                d.wait()

    return gather


_PART_BASES = [sum(PARTS[:i]) for i in range(len(PARTS))]
_GATHERS = [_make_gather(b, n) for b, n in zip(_PART_BASES, PARTS)]


def _edge_body(g_ref, r_ref, s_ref, w2_ref, b2_ref, g2_ref,
               be2_ref, o_ref):
    g = g_ref[...].astype(jnp.bfloat16)
    a = jnp.dot(g[:, :DP], r_ref[...], preferred_element_type=jnp.float32)
    b = jnp.dot(g[:, DP:2 * DP], s_ref[...],
                preferred_element_type=jnp.float32)
    kron = (a * b).astype(jnp.bfloat16)
    y = jnp.dot(kron, w2_ref[...], preferred_element_type=jnp.float32)
    y = y + b2_ref[...]
    mu = jnp.mean(y, axis=1, keepdims=True)
    var = jnp.mean(y * y, axis=1, keepdims=True) - mu * mu
    h = (y - mu) * lax.rsqrt(var + 1e-5) * g2_ref[...] + be2_ref[...]
    o_ref[...] = jnp.maximum(h, 0.0)


def _edge_mlp(g, rmat, smat, w2, b2, g2, be2, blk):
    e = g.shape[0]
    grid = e // blk
    return pl.pallas_call(
        _edge_body,
        grid=(grid,),
        in_specs=[
            pl.BlockSpec((blk, OUT), lambda i: (i, 0)),
            pl.BlockSpec((DP, KRON), lambda i: (0, 0)),
            pl.BlockSpec((DP, KRON), lambda i: (0, 0)),
            pl.BlockSpec((KRON, OUT), lambda i: (0, 0)),
            pl.BlockSpec((1, OUT), lambda i: (0, 0)),
            pl.BlockSpec((1, OUT), lambda i: (0, 0)),
            pl.BlockSpec((1, OUT), lambda i: (0, 0)),
        ],
        out_specs=pl.BlockSpec((blk, OUT), lambda i: (i, 0)),
        out_shape=jax.ShapeDtypeStruct((e, OUT), jnp.float32),
    )(g, rmat, smat, w2, b2, g2, be2)


def _make_scatter(base_ch, count):
    kc = count // NC
    tc = kc // NS
    n_extra_t = kc - NS * tc

    @functools.partial(
        pl.kernel,
        out_type=(
            jax.ShapeDtypeStruct((NP, OUT), jnp.float32),
            jax.ShapeDtypeStruct((NP, OUT), jnp.float32),
        ),
        mesh=_MESH,
        scratch_types=[
            pltpu.VMEM_SHARED((NP, OUT), jnp.float32),
            pltpu.VMEM((2, CH, OUT), jnp.float32),
            pltpu.VMEM((tc + 1, CH), jnp.int32),
            pltpu.SemaphoreType.DMA,
            pltpu.SemaphoreType.DMA,
        ],
        compiler_params=pltpu.CompilerParams(use_tc_tiling_on_sc=False),
    )
    def scatter(m_hbm, eid_hbm, zrows_hbm, p0_hbm, p1_hbm,
                acc, mv, idxv, sem_m0, sem_m1):
        c = lax.axis_index("c")
        s = lax.axis_index("s")
        loc0 = c * kc + s * tc
        sems = (sem_m0, sem_m1)

        pltpu.sync_copy(eid_hbm.at[pl.ds(base_ch + loc0, tc)],
                        idxv.at[pl.ds(0, tc)])
        if n_extra_t:
            @pl.when(s < n_extra_t)
            def _():
                ex = c * kc + NS * tc + s
                pltpu.sync_copy(eid_hbm.at[pl.ds(base_ch + ex, 1)],
                                idxv.at[pl.ds(tc, 1)])

        pltpu.sync_copy(zrows_hbm, mv.at[0])
        for j in range(ROWS_PER_TILE // CH):
            pltpu.sync_copy(mv.at[0],
                            acc.at[pl.ds(s * ROWS_PER_TILE + j * CH, CH)])
        plsc.subcore_barrier()

        pending = {0: pltpu.async_copy(
            m_hbm.at[pl.ds(loc0 * CH, CH)], mv.at[0], sem_m0)}
        for i in range(tc):
            b = i % 2
            if i + 1 < tc:
                pending[1 - b] = pltpu.async_copy(
                    m_hbm.at[pl.ds((loc0 + i + 1) * CH, CH)], mv.at[1 - b],
                    sems[1 - b])
            pending.pop(b).wait()
            pltpu.sync_copy(mv.at[b], acc.at[idxv.at[i]], add=True)

        if n_extra_t:
            @pl.when(s < n_extra_t)
            def _():
                ex = c * kc + NS * tc + s
                pltpu.sync_copy(m_hbm.at[pl.ds(ex * CH, CH)], mv.at[0])
                pltpu.sync_copy(mv.at[0], acc.at[idxv.at[tc]], add=True)

        plsc.subcore_barrier()
        for j in range(ROWS_PER_TILE // CH):
            row = s * ROWS_PER_TILE + j * CH
            pltpu.sync_copy(acc.at[pl.ds(row, CH)], mv.at[j % 2])

            @pl.when(c == 0)
            def _():
                pltpu.sync_copy(mv.at[j % 2], p0_hbm.at[pl.ds(row, CH)])

            @pl.when(c == 1)
            def _():
                pltpu.sync_copy(mv.at[j % 2], p1_hbm.at[pl.ds(row, CH)])

    return scatter


_SCATTERS = [_make_scatter(b, n) for b, n in zip(_PART_BASES, PARTS)]


def _combine_body(*refs):
    o_ref = refs[-1]
    acc = refs[0][...]
    for r in refs[1:-1]:
        acc = acc + r[...]
    o_ref[...] = acc


def _combine(partials):
    blk = 2000
    grid = N // blk
    spec = pl.BlockSpec((blk, OUT), lambda i: (i, 0))
    return pl.pallas_call(
        _combine_body,
        grid=(grid,),
        in_specs=[spec] * len(partials),
        out_specs=spec,
        out_shape=jax.ShapeDtypeStruct((N, OUT), jnp.float32),
    )(*partials)


def _build_rs():
    r = np.zeros((DP, KRON), np.float32)
    s = np.zeros((DP, KRON), np.float32)
    for a in range(20):
        for k in range(20):
            r[a, a * 20 + k] = 1.0
            s[k, a * 20 + k] = 1.0
    return r, s


_R_NP, _S_NP = _build_rs()


def kernel(node_feat, edge_index, W1, b1, g1, be1, W2, b2, g2, be2):
    w1p = jnp.pad(W1, ((0, 0), (0, DP - 20)))
    b1p = jnp.pad(b1, (0, DP - 20)).reshape(1, DP)
    g1p = jnp.pad(g1, (0, DP - 20)).reshape(1, DP)
    be1p = jnp.pad(be1, (0, DP - 20)).reshape(1, DP)
    ei_src = edge_index[0].reshape(NCHUNK, CH)
    ei_dst = edge_index[1].reshape(NCHUNK, CH)
    zrows = jnp.zeros((CH, OUT), jnp.float32)
    rmat = jnp.asarray(_R_NP, dtype=jnp.bfloat16)
    smat = jnp.asarray(_S_NP, dtype=jnp.bfloat16)
    w2b = W2.astype(jnp.bfloat16)
    b2r = b2.reshape(1, OUT)
    g2r = g2.reshape(1, OUT)
    be2r = be2.reshape(1, OUT)

    h32 = _node_proj(node_feat, w1p, b1p, g1p, be1p)
    gs = [gk(h32, ei_src, ei_dst) for gk in _GATHERS]
    ms = [_edge_mlp(g, rmat, smat, w2b, b2r, g2r, be2r,
                    blk=g.shape[0] // 8) for g in gs]
    partials = []
    for sk, m in zip(_SCATTERS, ms):
        partials.extend(sk(m, ei_dst, zrows))
    return _combine(partials)
